# Initial kernel scaffold; baseline (speedup 1.0000x reference)
import jax
import jax.numpy as jnp
from jax.experimental import pallas as pl


def kernel(audio, params):
    raise NotImplementedError("write your pallas kernel here")


# speedup vs baseline: 1.0063x; 1.0063x over previous
import functools

import jax
import jax.numpy as jnp
from jax import lax
from jax.experimental import pallas as pl
from jax.experimental.pallas import tpu as pltpu


def _conv1d(x, W, b, stride=1):
    out = lax.conv_general_dilated(
        x, W, window_strides=(stride,), padding=[(0, 0)],
        dimension_numbers=('NCH', 'OIH', 'NCH'))
    return out + b[None, :, None]


def _causal_conv1d(x, W, b, stride=1):
    K = W.shape[2]
    pad = (K - 1) * stride
    x = jnp.pad(x, ((0, 0), (0, 0), (pad, 0)))
    return _conv1d(x, W, b, stride)


def _group_norm(x, g, b, eps=1e-5):
    mu = jnp.mean(x, axis=(1, 2), keepdims=True)
    var = jnp.var(x, axis=(1, 2), keepdims=True)
    xn = (x - mu) / jnp.sqrt(var + eps)
    return xn * g[None, :, None] + b[None, :, None]


def _silu(x):
    return x * jax.nn.sigmoid(x)


def _res_block(x, p):
    r = x
    h = _group_norm(x, p['g1'], p['b1'])
    h = _silu(h)
    h = _causal_conv1d(h, p['W1'], p['bc1'])
    h = _group_norm(h, p['g2'], p['b2'])
    h = _silu(h)
    h = _causal_conv1d(h, p['W2'], p['bc2'])
    return h + r


def _conv_transpose1d(x, W, b, stride, padding):
    K = W.shape[2]
    Wf = jnp.flip(W, axis=2).transpose(1, 0, 2)
    out = lax.conv_general_dilated(
        x, Wf, window_strides=(1,), padding=[(K - 1 - padding, K - 1 - padding)],
        lhs_dilation=(stride,), dimension_numbers=('NCH', 'OIH', 'NCH'))
    return out + b[None, :, None]


def _encoder_fwd(audio, p):
    x = _causal_conv1d(audio, p['in_W'], p['in_b'])
    for bp in p['blocks']:
        x = _causal_conv1d(x, bp['down_W'], bp['down_b'], stride=2)
        x = _group_norm(x, bp['gn_g'], bp['gn_b'])
        x = _silu(x)
        for rp in bp['res']:
            x = _res_block(x, rp)
    sem = jnp.tanh(_group_norm(_conv1d(x, p['sem_W'], p['sem_b']),
                               p['sem_g'], p['sem_gb']))
    ac = _group_norm(_conv1d(x, p['ac_W'], p['ac_b']), p['ac_g'], p['ac_gb'])
    return jnp.concatenate([sem, ac], axis=1)


def _decoder_tail(x, p):
    for bp in p['blocks']:
        for rp in bp['res']:
            x = _res_block(x, rp)
        x = _conv_transpose1d(x, bp['up_W'], bp['up_b'], stride=2, padding=1)
        x = _group_norm(x, bp['gn_g'], bp['gn_b'])
        x = _silu(x)
    x = jnp.pad(x, ((0, 0), (0, 0), (3, 3)))
    x = _conv1d(x, p['out_W'], p['out_b'])
    return jnp.tanh(x)


def _rvq_body(xT_ref, cb_ref, cbn_ref, hi_ref, mid_ref, lo_ref,
              out_ref, r_ref, q_ref, *, n_cb):
    c = pl.program_id(1)
    bf16 = jnp.bfloat16
    f32 = jnp.float32

    @pl.when(c == 0)
    def _init():
        r_ref[...] = xT_ref[...]
        q_ref[...] = jnp.zeros_like(q_ref)

    r = r_ref[...]
    rn = jnp.sum(r * r, axis=0, keepdims=True)
    s = lax.dot_general(cb_ref[0], r.astype(bf16),
                        (((1,), (0,)), ((), ())),
                        preferred_element_type=f32)
    d = (rn - 2.0 * s) + cbn_ref[0]
    m = jnp.min(d, axis=0, keepdims=True)
    K = d.shape[0]
    iota = lax.broadcasted_iota(jnp.int32, d.shape, 0)
    sel = jnp.where(d == m, iota, K)
    idx = jnp.min(sel, axis=0, keepdims=True)
    oh = (iota == idx).astype(bf16)
    cn = (((1,), (0,)), ((), ()))
    qstep = ((lax.dot_general(hi_ref[0], oh, cn, preferred_element_type=f32)
              + lax.dot_general(mid_ref[0], oh, cn, preferred_element_type=f32))
             + lax.dot_general(lo_ref[0], oh, cn, preferred_element_type=f32))
    r_ref[...] = r - qstep
    q_ref[...] = q_ref[...] + qstep

    @pl.when(c == n_cb - 1)
    def _fin():
        out_ref[...] = q_ref[...]


_TILE = 256


@functools.partial(jax.jit, static_argnames=('interpret',))
def _rvq_pallas(x, vq_params, interpret=False):
    B, T, D = x.shape
    BT = B * T
    cbs = vq_params['codebooks']
    n_cb, K, _ = cbs.shape

    cbn = jnp.sum(cbs ** 2, axis=-1)[..., None]
    cb_bf = cbs.astype(jnp.bfloat16)

    def _trunc_bf16(v):
        bits = lax.bitcast_convert_type(v, jnp.uint32)
        return lax.bitcast_convert_type(bits & jnp.uint32(0xFFFF0000),
                                        jnp.float32)

    hi_f = _trunc_bf16(cbs)
    r1 = cbs - hi_f
    mid_f = _trunc_bf16(r1)
    lo_f = r1 - mid_f
    hiT = hi_f.astype(jnp.bfloat16).transpose(0, 2, 1)
    midT = mid_f.astype(jnp.bfloat16).transpose(0, 2, 1)
    loT = lo_f.astype(jnp.bfloat16).transpose(0, 2, 1)

    xT = x.reshape(BT, D).T

    ntiles = pl.cdiv(BT, _TILE)
    BTp = ntiles * _TILE
    if BTp != BT:
        xT = jnp.pad(xT, ((0, 0), (0, BTp - BT)))

    out = pl.pallas_call(
        functools.partial(_rvq_body, n_cb=n_cb),
        grid=(ntiles, n_cb),
        in_specs=[
            pl.BlockSpec((D, _TILE), lambda rt, c: (0, rt)),
            pl.BlockSpec((1, K, D), lambda rt, c: (c, 0, 0)),
            pl.BlockSpec((1, K, 1), lambda rt, c: (c, 0, 0)),
            pl.BlockSpec((1, D, K), lambda rt, c: (c, 0, 0)),
            pl.BlockSpec((1, D, K), lambda rt, c: (c, 0, 0)),
            pl.BlockSpec((1, D, K), lambda rt, c: (c, 0, 0)),
        ],
        out_specs=pl.BlockSpec((D, _TILE), lambda rt, c: (0, rt)),
        out_shape=jax.ShapeDtypeStruct((D, BTp), jnp.float32),
        scratch_shapes=[pltpu.VMEM((D, _TILE), jnp.float32),
                        pltpu.VMEM((D, _TILE), jnp.float32)],
        interpret=interpret,
    )(xT, cb_bf, cbn, hiT, midT, loT)

    return out[:, :BT].T.reshape(B, T, D)


def kernel(audio, params):
    latent = _encoder_fwd(audio, params['enc'])
    vq = params['vq']
    x = _conv1d(latent, vq['proj_W'], vq['proj_b']).transpose(0, 2, 1)
    q = _rvq_pallas(x, vq)
    quantized = (x + lax.stop_gradient(q - x)).transpose(0, 2, 1)
    dec = params['dec']
    dx = _conv1d(quantized, dec['in_W'], dec['in_b'])
    return _decoder_tail(dx, dec)


# speedup vs baseline: 1.0365x; 1.0300x over previous
import functools

import jax
import jax.numpy as jnp
from jax import lax
from jax.experimental import pallas as pl
from jax.experimental.pallas import tpu as pltpu


def _conv1d(x, W, b, stride=1):
    out = lax.conv_general_dilated(
        x, W, window_strides=(stride,), padding=[(0, 0)],
        dimension_numbers=('NCH', 'OIH', 'NCH'))
    return out + b[None, :, None]


def _causal_conv1d(x, W, b, stride=1):
    K = W.shape[2]
    pad = (K - 1) * stride
    x = jnp.pad(x, ((0, 0), (0, 0), (pad, 0)))
    return _conv1d(x, W, b, stride)


def _group_norm(x, g, b, eps=1e-5):
    mu = jnp.mean(x, axis=(1, 2), keepdims=True)
    var = jnp.var(x, axis=(1, 2), keepdims=True)
    xn = (x - mu) / jnp.sqrt(var + eps)
    return xn * g[None, :, None] + b[None, :, None]


def _silu(x):
    return x * jax.nn.sigmoid(x)


def _res_block(x, p):
    r = x
    h = _group_norm(x, p['g1'], p['b1'])
    h = _silu(h)
    h = _causal_conv1d(h, p['W1'], p['bc1'])
    h = _group_norm(h, p['g2'], p['b2'])
    h = _silu(h)
    h = _causal_conv1d(h, p['W2'], p['bc2'])
    return h + r


def _conv_transpose1d(x, W, b, stride, padding):
    K = W.shape[2]
    Wf = jnp.flip(W, axis=2).transpose(1, 0, 2)
    out = lax.conv_general_dilated(
        x, Wf, window_strides=(1,), padding=[(K - 1 - padding, K - 1 - padding)],
        lhs_dilation=(stride,), dimension_numbers=('NCH', 'OIH', 'NCH'))
    return out + b[None, :, None]


def _encoder_fwd(audio, p):
    x = _causal_conv1d(audio, p['in_W'], p['in_b'])
    for bp in p['blocks']:
        x = _causal_conv1d(x, bp['down_W'], bp['down_b'], stride=2)
        x = _group_norm(x, bp['gn_g'], bp['gn_b'])
        x = _silu(x)
        for rp in bp['res']:
            x = _res_block(x, rp)
    sem = jnp.tanh(_group_norm(_conv1d(x, p['sem_W'], p['sem_b']),
                               p['sem_g'], p['sem_gb']))
    ac = _group_norm(_conv1d(x, p['ac_W'], p['ac_b']), p['ac_g'], p['ac_gb'])
    return jnp.concatenate([sem, ac], axis=1)


def _decoder_tail(x, p):
    for bp in p['blocks']:
        for rp in bp['res']:
            x = _res_block(x, rp)
        x = _conv_transpose1d(x, bp['up_W'], bp['up_b'], stride=2, padding=1)
        x = _group_norm(x, bp['gn_g'], bp['gn_b'])
        x = _silu(x)
    x = jnp.pad(x, ((0, 0), (0, 0), (3, 3)))
    x = _conv1d(x, p['out_W'], p['out_b'])
    return jnp.tanh(x)


def _rvq_body(xT_ref, cb_ref, cbn_ref, hi_ref, mid_ref, lo_ref,
              out_ref, r_ref, q_ref, *, n_cb):
    c = pl.program_id(1)
    bf16 = jnp.bfloat16
    f32 = jnp.float32

    @pl.when(c == 0)
    def _init():
        r_ref[...] = xT_ref[...]
        q_ref[...] = jnp.zeros_like(q_ref)

    r = r_ref[...]
    rn = jnp.sum(r * r, axis=0, keepdims=True)
    s = lax.dot_general(cb_ref[0], r.astype(bf16),
                        (((1,), (0,)), ((), ())),
                        preferred_element_type=f32)
    d = (rn - 2.0 * s) + cbn_ref[0]
    m = jnp.min(d, axis=0, keepdims=True)
    K = d.shape[0]
    iota = lax.broadcasted_iota(jnp.int32, d.shape, 0)
    sel = jnp.where(d == m, iota, K)
    idx = jnp.min(sel, axis=0, keepdims=True)
    oh = (iota == idx).astype(bf16)
    cn = (((1,), (0,)), ((), ()))
    qstep = ((lax.dot_general(hi_ref[0], oh, cn, preferred_element_type=f32)
              + lax.dot_general(mid_ref[0], oh, cn, preferred_element_type=f32))
             + lax.dot_general(lo_ref[0], oh, cn, preferred_element_type=f32))
    r_ref[...] = r - qstep
    q_ref[...] = q_ref[...] + qstep

    @pl.when(c == n_cb - 1)
    def _fin():
        out_ref[...] = q_ref[...]


_TILE = 512


@functools.partial(jax.jit, static_argnames=('interpret',))
def _rvq_pallas(x, vq_params, interpret=False):
    B, T, D = x.shape
    BT = B * T
    cbs = vq_params['codebooks']
    n_cb, K, _ = cbs.shape

    cbn = jnp.sum(cbs ** 2, axis=-1)[..., None]
    cb_bf = cbs.astype(jnp.bfloat16)

    def _trunc_bf16(v):
        bits = lax.bitcast_convert_type(v, jnp.uint32)
        return lax.bitcast_convert_type(bits & jnp.uint32(0xFFFF0000),
                                        jnp.float32)

    hi_f = _trunc_bf16(cbs)
    r1 = cbs - hi_f
    mid_f = _trunc_bf16(r1)
    lo_f = r1 - mid_f
    hiT = hi_f.astype(jnp.bfloat16).transpose(0, 2, 1)
    midT = mid_f.astype(jnp.bfloat16).transpose(0, 2, 1)
    loT = lo_f.astype(jnp.bfloat16).transpose(0, 2, 1)

    xT = x.reshape(BT, D).T

    ntiles = pl.cdiv(BT, _TILE)
    BTp = ntiles * _TILE
    if BTp != BT:
        xT = jnp.pad(xT, ((0, 0), (0, BTp - BT)))

    out = pl.pallas_call(
        functools.partial(_rvq_body, n_cb=n_cb),
        grid=(ntiles, n_cb),
        in_specs=[
            pl.BlockSpec((D, _TILE), lambda rt, c: (0, rt)),
            pl.BlockSpec((1, K, D), lambda rt, c: (c, 0, 0)),
            pl.BlockSpec((1, K, 1), lambda rt, c: (c, 0, 0)),
            pl.BlockSpec((1, D, K), lambda rt, c: (c, 0, 0)),
            pl.BlockSpec((1, D, K), lambda rt, c: (c, 0, 0)),
            pl.BlockSpec((1, D, K), lambda rt, c: (c, 0, 0)),
        ],
        out_specs=pl.BlockSpec((D, _TILE), lambda rt, c: (0, rt)),
        out_shape=jax.ShapeDtypeStruct((D, BTp), jnp.float32),
        scratch_shapes=[pltpu.VMEM((D, _TILE), jnp.float32),
                        pltpu.VMEM((D, _TILE), jnp.float32)],
        interpret=interpret,
    )(xT, cb_bf, cbn, hiT, midT, loT)

    return out[:, :BT].T.reshape(B, T, D)


def kernel(audio, params):
    latent = _encoder_fwd(audio, params['enc'])
    vq = params['vq']
    x = _conv1d(latent, vq['proj_W'], vq['proj_b']).transpose(0, 2, 1)
    q = _rvq_pallas(x, vq)
    quantized = (x + lax.stop_gradient(q - x)).transpose(0, 2, 1)
    dec = params['dec']
    dx = _conv1d(quantized, dec['in_W'], dec['in_b'])
    return _decoder_tail(dx, dec)


# speedup vs baseline: 1.0483x; 1.0114x over previous
import functools

import jax
import jax.numpy as jnp
from jax import lax
from jax.experimental import pallas as pl
from jax.experimental.pallas import tpu as pltpu


def _conv1d(x, W, b, stride=1):
    out = lax.conv_general_dilated(
        x, W, window_strides=(stride,), padding=[(0, 0)],
        dimension_numbers=('NCH', 'OIH', 'NCH'))
    return out + b[None, :, None]


def _causal_conv1d(x, W, b, stride=1):
    K = W.shape[2]
    pad = (K - 1) * stride
    x = jnp.pad(x, ((0, 0), (0, 0), (pad, 0)))
    return _conv1d(x, W, b, stride)


def _group_norm(x, g, b, eps=1e-5):
    mu = jnp.mean(x, axis=(1, 2), keepdims=True)
    var = jnp.var(x, axis=(1, 2), keepdims=True)
    xn = (x - mu) / jnp.sqrt(var + eps)
    return xn * g[None, :, None] + b[None, :, None]


def _silu(x):
    return x * jax.nn.sigmoid(x)


def _res_block(x, p):
    r = x
    h = _group_norm(x, p['g1'], p['b1'])
    h = _silu(h)
    h = _causal_conv1d(h, p['W1'], p['bc1'])
    h = _group_norm(h, p['g2'], p['b2'])
    h = _silu(h)
    h = _causal_conv1d(h, p['W2'], p['bc2'])
    return h + r


def _conv_transpose1d(x, W, b, stride, padding):
    K = W.shape[2]
    Wf = jnp.flip(W, axis=2).transpose(1, 0, 2)
    out = lax.conv_general_dilated(
        x, Wf, window_strides=(1,), padding=[(K - 1 - padding, K - 1 - padding)],
        lhs_dilation=(stride,), dimension_numbers=('NCH', 'OIH', 'NCH'))
    return out + b[None, :, None]


def _encoder_fwd(audio, p):
    x = _causal_conv1d(audio, p['in_W'], p['in_b'])
    for bp in p['blocks']:
        x = _causal_conv1d(x, bp['down_W'], bp['down_b'], stride=2)
        x = _group_norm(x, bp['gn_g'], bp['gn_b'])
        x = _silu(x)
        for rp in bp['res']:
            x = _res_block(x, rp)
    sem = jnp.tanh(_group_norm(_conv1d(x, p['sem_W'], p['sem_b']),
                               p['sem_g'], p['sem_gb']))
    ac = _group_norm(_conv1d(x, p['ac_W'], p['ac_b']), p['ac_g'], p['ac_gb'])
    return jnp.concatenate([sem, ac], axis=1)


def _decoder_tail(x, p):
    for bp in p['blocks']:
        for rp in bp['res']:
            x = _res_block(x, rp)
        x = _conv_transpose1d(x, bp['up_W'], bp['up_b'], stride=2, padding=1)
        x = _group_norm(x, bp['gn_g'], bp['gn_b'])
        x = _silu(x)
    x = jnp.pad(x, ((0, 0), (0, 0), (3, 3)))
    x = _conv1d(x, p['out_W'], p['out_b'])
    return jnp.tanh(x)


def _rvq_body(xT_ref, cb_ref, cbn_ref, hi_ref, mid_ref, lo_ref,
              out_ref, r_ref, q_ref, *, n_cb):
    c = pl.program_id(1)
    bf16 = jnp.bfloat16
    f32 = jnp.float32

    @pl.when(c == 0)
    def _init():
        r_ref[...] = xT_ref[...]
        q_ref[...] = jnp.zeros_like(q_ref)

    r = r_ref[...]
    rn = jnp.sum(r * r, axis=0, keepdims=True)
    s = lax.dot_general(cb_ref[0], r.astype(bf16),
                        (((1,), (0,)), ((), ())),
                        preferred_element_type=f32)
    d = (rn - 2.0 * s) + cbn_ref[0]
    m = jnp.min(d, axis=0, keepdims=True)
    K = d.shape[0]
    iota = lax.broadcasted_iota(jnp.int32, d.shape, 0)
    sel = jnp.where(d == m, iota, K)
    idx = jnp.min(sel, axis=0, keepdims=True)
    oh = (iota == idx).astype(bf16)
    cn = (((1,), (0,)), ((), ()))
    qstep = ((lax.dot_general(hi_ref[0], oh, cn, preferred_element_type=f32)
              + lax.dot_general(mid_ref[0], oh, cn, preferred_element_type=f32))
             + lax.dot_general(lo_ref[0], oh, cn, preferred_element_type=f32))
    r_ref[...] = r - qstep
    q_ref[...] = q_ref[...] + qstep

    @pl.when(c == n_cb - 1)
    def _fin():
        out_ref[...] = q_ref[...]


_TILE = 768


@functools.partial(jax.jit, static_argnames=('interpret',))
def _rvq_pallas(x, vq_params, interpret=False):
    B, T, D = x.shape
    BT = B * T
    cbs = vq_params['codebooks']
    n_cb, K, _ = cbs.shape

    cbn = jnp.sum(cbs ** 2, axis=-1)[..., None]
    cb_bf = cbs.astype(jnp.bfloat16)

    def _trunc_bf16(v):
        bits = lax.bitcast_convert_type(v, jnp.uint32)
        return lax.bitcast_convert_type(bits & jnp.uint32(0xFFFF0000),
                                        jnp.float32)

    hi_f = _trunc_bf16(cbs)
    r1 = cbs - hi_f
    mid_f = _trunc_bf16(r1)
    lo_f = r1 - mid_f
    hiT = hi_f.astype(jnp.bfloat16).transpose(0, 2, 1)
    midT = mid_f.astype(jnp.bfloat16).transpose(0, 2, 1)
    loT = lo_f.astype(jnp.bfloat16).transpose(0, 2, 1)

    xT = x.reshape(BT, D).T

    ntiles = pl.cdiv(BT, _TILE)
    BTp = ntiles * _TILE
    if BTp != BT:
        xT = jnp.pad(xT, ((0, 0), (0, BTp - BT)))

    out = pl.pallas_call(
        functools.partial(_rvq_body, n_cb=n_cb),
        grid=(ntiles, n_cb),
        in_specs=[
            pl.BlockSpec((D, _TILE), lambda rt, c: (0, rt)),
            pl.BlockSpec((1, K, D), lambda rt, c: (c, 0, 0)),
            pl.BlockSpec((1, K, 1), lambda rt, c: (c, 0, 0)),
            pl.BlockSpec((1, D, K), lambda rt, c: (c, 0, 0)),
            pl.BlockSpec((1, D, K), lambda rt, c: (c, 0, 0)),
            pl.BlockSpec((1, D, K), lambda rt, c: (c, 0, 0)),
        ],
        out_specs=pl.BlockSpec((D, _TILE), lambda rt, c: (0, rt)),
        out_shape=jax.ShapeDtypeStruct((D, BTp), jnp.float32),
        scratch_shapes=[pltpu.VMEM((D, _TILE), jnp.float32),
                        pltpu.VMEM((D, _TILE), jnp.float32)],
        interpret=interpret,
    )(xT, cb_bf, cbn, hiT, midT, loT)

    return out[:, :BT].T.reshape(B, T, D)


def kernel(audio, params):
    latent = _encoder_fwd(audio, params['enc'])
    vq = params['vq']
    x = _conv1d(latent, vq['proj_W'], vq['proj_b']).transpose(0, 2, 1)
    q = _rvq_pallas(x, vq)
    quantized = (x + lax.stop_gradient(q - x)).transpose(0, 2, 1)
    dec = params['dec']
    dx = _conv1d(quantized, dec['in_W'], dec['in_b'])
    return _decoder_tail(dx, dec)
